# baseline (speedup 1.0000x reference)
import functools

import jax
import jax.numpy as jnp
from jax import lax
from jax.experimental import pallas as pl
from jax.experimental.pallas import tpu as pltpu
from jax.experimental.pallas import tpu_sc as plsc

VOCAB = 100002
EMBED = 100
BATCH = 4096
SEQ = 200

NC = 2
NS = 16
NW = NC * NS

FULL_DIMS = EMBED // NW * NW
REM_DIMS = EMBED - FULL_DIMS
K_FULL = FULL_DIMS // NW

BR = 8
BC = 512
NB_S = SEQ // BR
NB_B = BATCH // BC
NBLK = NB_S * NB_B
TILES_PER_REM = NW // REM_DIMS
NBLK_REM = NBLK // TILES_PER_REM

VPB = (BR * BC) // 16


QCOLS = 1024
NPHASE = BATCH // QCOLS
NB_BQ = QCOLS // BC
NBLK_Q = NB_S * NB_BQ


def _body(idx_hbm, tab_hbm, out_hbm, row_v, ibufs, obufs, isems, osems,
          idx_sp):
    cid = lax.axis_index("c")
    sid = lax.axis_index("s")
    wid = sid * NC + cid

    def gather_block(p):
        ib, ob = ibufs[p], obufs[p]
        for r in range(BR):
            @plsc.parallel_loop(0, BC, step=16, unroll=16)
            def _vec(c):
                ix = ib[r, pl.ds(c, 16)]
                ob[r, pl.ds(c, 16)] = plsc.load_gather(row_v, [ix])

    for q in range(NPHASE):
        @pl.when(sid == 0)
        def _():
            pltpu.sync_copy(
                idx_hbm.at[:, pl.ds(q * QCOLS, QCOLS)], idx_sp
            )

        plsc.subcore_barrier()

        def idx_src(bl):
            sr = lax.div(bl, NB_BQ) * BR
            bc = lax.rem(bl, NB_BQ) * BC
            return idx_sp.at[pl.ds(sr, BR), pl.ds(bc, BC)]

        def out_dst(e, bl):
            sr = lax.div(bl, NB_BQ) * BR
            bc = lax.rem(bl, NB_BQ) * BC + q * QCOLS
            return out_hbm.at[e, pl.ds(sr, BR), pl.ds(bc, BC)]

        def start_idx(bl, p):
            pltpu.async_copy(idx_src(bl), ibufs[p], isems[p])

        def wait_idx(bl, p):
            pltpu.make_async_copy(idx_src(bl), ibufs[p], isems[p]).wait()

        def start_out(e, bl, p):
            pltpu.async_copy(obufs[p], out_dst(e, bl), osems[p])

        def wait_out(e, bl, p):
            pltpu.make_async_copy(obufs[p], out_dst(e, bl), osems[p]).wait()

        def do_dim(e):
            pltpu.sync_copy(tab_hbm.at[e], row_v)
            start_idx(0, 0)
            start_idx(1, 1)

            def step(bl, p):
                wait_idx(bl, p)
                gather_block(p)

                @pl.when(bl + 2 < NBLK_Q)
                def _():
                    start_idx(bl + 2, p)

                @pl.when(bl - 2 >= 0)
                def _():
                    wait_out(e, bl - 2, p)

                start_out(e, bl, p)

            def pair(i, carry):
                step(2 * i, 0)
                step(2 * i + 1, 1)
                return carry

            lax.fori_loop(0, NBLK_Q // 2, pair, 0)
            wait_out(e, NBLK_Q - 1, 1)
            wait_out(e, NBLK_Q - 2, 0)

        def full_dim(k, carry):
            do_dim(wid + NW * k)
            return carry

        lax.fori_loop(0, K_FULL, full_dim, 0)

        @pl.when((wid >= REM_DIMS * q) & (wid < REM_DIMS * (q + 1)))
        def _():
            do_dim(FULL_DIMS + wid - REM_DIMS * q)

        plsc.subcore_barrier()


@functools.partial(jax.jit, static_argnums=())
def kernel(indices, table):
    idx_t = indices.astype(jnp.int32).T
    tab_t = table.T
    mesh = plsc.VectorSubcoreMesh(core_axis_name="c", subcore_axis_name="s")
    run = pl.kernel(
        _body,
        mesh=mesh,
        compiler_params=pltpu.CompilerParams(needs_layout_passes=False),
        out_type=jax.ShapeDtypeStruct((EMBED, SEQ, BATCH), jnp.float32),
        scratch_types=[
            pltpu.VMEM((VOCAB,), jnp.float32),
            [pltpu.VMEM((BR, BC), jnp.int32) for _ in range(2)],
            [pltpu.VMEM((BR, BC), jnp.float32) for _ in range(2)],
            [pltpu.SemaphoreType.DMA for _ in range(2)],
            [pltpu.SemaphoreType.DMA for _ in range(2)],
            pltpu.VMEM_SHARED((SEQ, QCOLS), jnp.int32),
        ],
    )
    out_t = run(idx_t, tab_t)
    return jnp.transpose(out_t, (2, 1, 0))


# speedup vs baseline: 1.0609x; 1.0609x over previous
import functools

import jax
import jax.numpy as jnp
from jax import lax
from jax.experimental import pallas as pl
from jax.experimental.pallas import tpu as pltpu
from jax.experimental.pallas import tpu_sc as plsc

VOCAB = 100002
EMBED = 100
BATCH = 4096
SEQ = 200

NC = 2
NS = 16
NW = NC * NS

FULL_DIMS = EMBED // NW * NW
REM_DIMS = EMBED - FULL_DIMS
K_FULL = FULL_DIMS // NW

BR = 8
BC = 512
NB_S = SEQ // BR
NB_B = BATCH // BC
NBLK = NB_S * NB_B
TILES_PER_REM = NW // REM_DIMS
NBLK_REM = NBLK // TILES_PER_REM

VPB = (BR * BC) // 16


QCOLS = 1024
NPHASE = BATCH // QCOLS
NB_BQ = QCOLS // BC
NBLK_Q = NB_S * NB_BQ


def _body(idx_hbm, tab_hbm, out_hbm, row_v, ibufs, obufs, isems, osems,
          idx_sp):
    cid = lax.axis_index("c")
    sid = lax.axis_index("s")
    wid = sid * NC + cid

    def gather_block(p):
        ib, ob = ibufs[p], obufs[p]
        for r in range(BR):
            @plsc.parallel_loop(0, BC, step=16, unroll=8)
            def _vec(c):
                ix = ib[r, pl.ds(c, 16)]
                ob[r, pl.ds(c, 16)] = plsc.load_gather(row_v, [ix])

    for q in range(NPHASE):
        @pl.when(sid == 0)
        def _():
            pltpu.sync_copy(
                idx_hbm.at[:, pl.ds(q * QCOLS, QCOLS)], idx_sp
            )

        plsc.subcore_barrier()

        def idx_src(bl):
            sr = lax.div(bl, NB_BQ) * BR
            bc = lax.rem(bl, NB_BQ) * BC
            return idx_sp.at[pl.ds(sr, BR), pl.ds(bc, BC)]

        def out_dst(e, bl):
            sr = lax.div(bl, NB_BQ) * BR
            bc = lax.rem(bl, NB_BQ) * BC + q * QCOLS
            return out_hbm.at[e, pl.ds(sr, BR), pl.ds(bc, BC)]

        def start_idx(bl, p):
            pltpu.async_copy(idx_src(bl), ibufs[p], isems[p])

        def wait_idx(bl, p):
            pltpu.make_async_copy(idx_src(bl), ibufs[p], isems[p]).wait()

        def start_out(e, bl, p):
            pltpu.async_copy(obufs[p], out_dst(e, bl), osems[p])

        def wait_out(e, bl, p):
            pltpu.make_async_copy(obufs[p], out_dst(e, bl), osems[p]).wait()

        def do_dim(e):
            pltpu.sync_copy(tab_hbm.at[e], row_v)
            start_idx(0, 0)
            start_idx(1, 1)

            def step(bl, p):
                wait_idx(bl, p)
                gather_block(p)

                @pl.when(bl + 2 < NBLK_Q)
                def _():
                    start_idx(bl + 2, p)

                @pl.when(bl - 2 >= 0)
                def _():
                    wait_out(e, bl - 2, p)

                start_out(e, bl, p)

            def pair(i, carry):
                step(2 * i, 0)
                step(2 * i + 1, 1)
                return carry

            lax.fori_loop(0, NBLK_Q // 2, pair, 0)
            wait_out(e, NBLK_Q - 1, 1)
            wait_out(e, NBLK_Q - 2, 0)

        def full_dim(k, carry):
            do_dim(wid + NW * k)
            return carry

        lax.fori_loop(0, K_FULL, full_dim, 0)

        @pl.when((wid >= REM_DIMS * q) & (wid < REM_DIMS * (q + 1)))
        def _():
            do_dim(FULL_DIMS + wid - REM_DIMS * q)

        plsc.subcore_barrier()


@functools.partial(jax.jit, static_argnums=())
def kernel(indices, table):
    idx_t = indices.astype(jnp.int32).T
    tab_t = table.T
    mesh = plsc.VectorSubcoreMesh(core_axis_name="c", subcore_axis_name="s")
    run = pl.kernel(
        _body,
        mesh=mesh,
        compiler_params=pltpu.CompilerParams(needs_layout_passes=False),
        out_type=jax.ShapeDtypeStruct((EMBED, SEQ, BATCH), jnp.float32),
        scratch_types=[
            pltpu.VMEM((VOCAB,), jnp.float32),
            [pltpu.VMEM((BR, BC), jnp.int32) for _ in range(2)],
            [pltpu.VMEM((BR, BC), jnp.float32) for _ in range(2)],
            [pltpu.SemaphoreType.DMA for _ in range(2)],
            [pltpu.SemaphoreType.DMA for _ in range(2)],
            pltpu.VMEM_SHARED((SEQ, QCOLS), jnp.int32),
        ],
    )
    out_t = run(idx_t, tab_t)
    return jnp.transpose(out_t, (2, 1, 0))


# speedup vs baseline: 1.0732x; 1.0116x over previous
import functools

import jax
import jax.numpy as jnp
from jax import lax
from jax.experimental import pallas as pl
from jax.experimental.pallas import tpu as pltpu
from jax.experimental.pallas import tpu_sc as plsc

VOCAB = 100002
EMBED = 100
BATCH = 4096
SEQ = 200

NC = 2
NS = 16
NW = NC * NS

FULL_DIMS = EMBED // NW * NW
REM_DIMS = EMBED - FULL_DIMS
K_FULL = FULL_DIMS // NW

BR = 8
BC = 256
NB_S = SEQ // BR
NB_B = BATCH // BC
NBLK = NB_S * NB_B
TILES_PER_REM = NW // REM_DIMS
NBLK_REM = NBLK // TILES_PER_REM

VPB = (BR * BC) // 16


QCOLS = 1024
NPHASE = BATCH // QCOLS
NB_BQ = QCOLS // BC
NBLK_Q = NB_S * NB_BQ


RBC = 128
NRB_B = QCOLS // RBC
NRBLK = NB_S * NRB_B
TPD = NW // REM_DIMS
NRB_T = NRBLK // TPD


def _body(idx_hbm, tab_hbm, out_hbm, row_v, ibufs, obufs, isems, osems,
          ribufs, robufs, idx_sp):
    risems, rosems = isems, osems
    cid = lax.axis_index("c")
    sid = lax.axis_index("s")
    wid = sid * NC + cid

    def gather_block(p):
        ib, ob = ibufs[p], obufs[p]
        for r in range(BR):
            @plsc.parallel_loop(0, BC, step=16, unroll=8)
            def _vec(c):
                ix = ib[r, pl.ds(c, 16)]
                ob[r, pl.ds(c, 16)] = plsc.load_gather(row_v, [ix])

    def gather_rblock(p):
        ib, ob = ribufs[p], robufs[p]
        for r in range(BR):
            @plsc.parallel_loop(0, RBC, step=16, unroll=8)
            def _vec(c):
                ix = ib[r, pl.ds(c, 16)]
                ob[r, pl.ds(c, 16)] = plsc.load_gather(row_v, [ix])

    for q in range(NPHASE):
        @pl.when(sid == 0)
        def _():
            pltpu.sync_copy(
                idx_hbm.at[:, pl.ds(q * QCOLS, QCOLS)], idx_sp
            )

        plsc.subcore_barrier()

        def idx_src(bl):
            sr = lax.div(bl, NB_BQ) * BR
            bc = lax.rem(bl, NB_BQ) * BC
            return idx_sp.at[pl.ds(sr, BR), pl.ds(bc, BC)]

        def out_dst(e, bl):
            sr = lax.div(bl, NB_BQ) * BR
            bc = lax.rem(bl, NB_BQ) * BC + q * QCOLS
            return out_hbm.at[e, pl.ds(sr, BR), pl.ds(bc, BC)]

        def start_idx(bl, p):
            pltpu.async_copy(idx_src(bl), ibufs[p], isems[p])

        def wait_idx(bl, p):
            pltpu.make_async_copy(idx_src(bl), ibufs[p], isems[p]).wait()

        def start_out(e, bl, p):
            pltpu.async_copy(obufs[p], out_dst(e, bl), osems[p])

        def wait_out(e, bl, p):
            pltpu.make_async_copy(obufs[p], out_dst(e, bl), osems[p]).wait()

        def do_dim(e):
            pltpu.sync_copy(tab_hbm.at[e], row_v)
            start_idx(0, 0)
            start_idx(1, 1)

            def step(bl, p):
                wait_idx(bl, p)
                gather_block(p)

                @pl.when(bl + 2 < NBLK_Q)
                def _():
                    start_idx(bl + 2, p)

                @pl.when(bl - 2 >= 0)
                def _():
                    wait_out(e, bl - 2, p)

                start_out(e, bl, p)

            def pair(i, carry):
                step(2 * i, 0)
                step(2 * i + 1, 1)
                return carry

            lax.fori_loop(0, NBLK_Q // 2, pair, 0)
            wait_out(e, NBLK_Q - 1, 1)
            wait_out(e, NBLK_Q - 2, 0)

        def full_dim(k, carry):
            do_dim(wid + NW * k)
            return carry

        lax.fori_loop(0, K_FULL, full_dim, 0)

        def ridx_src(bl):
            sr = lax.div(bl, NRB_B) * BR
            bc = lax.rem(bl, NRB_B) * RBC
            return idx_sp.at[pl.ds(sr, BR), pl.ds(bc, RBC)]

        def rout_dst(e, bl):
            sr = lax.div(bl, NRB_B) * BR
            bc = lax.rem(bl, NRB_B) * RBC + q * QCOLS
            return out_hbm.at[e, pl.ds(sr, BR), pl.ds(bc, RBC)]

        def rstart_idx(bl, p):
            pltpu.async_copy(ridx_src(bl), ribufs[p], risems[p])

        def rwait_idx(bl, p):
            pltpu.make_async_copy(ridx_src(bl), ribufs[p], risems[p]).wait()

        def rstart_out(e, bl, p):
            pltpu.async_copy(robufs[p], rout_dst(e, bl), rosems[p])

        def rwait_out(e, bl, p):
            pltpu.make_async_copy(robufs[p], rout_dst(e, bl), rosems[p]).wait()

        e_rem = FULL_DIMS + lax.div(wid, TPD)
        lo_rem = lax.rem(wid, TPD) * NRB_T
        pltpu.sync_copy(tab_hbm.at[e_rem], row_v)
        rstart_idx(lo_rem, 0)
        rstart_idx(lo_rem + 1, 1)

        def rstep(bl, p):
            rwait_idx(bl, p)
            gather_rblock(p)

            @pl.when(bl + 2 < lo_rem + NRB_T)
            def _():
                rstart_idx(bl + 2, p)

            @pl.when(bl - 2 >= lo_rem)
            def _():
                rwait_out(e_rem, bl - 2, p)

            rstart_out(e_rem, bl, p)

        def rpair(i, carry):
            rstep(lo_rem + 2 * i, 0)
            rstep(lo_rem + 2 * i + 1, 1)
            return carry

        lax.fori_loop(0, NRB_T // 2, rpair, 0)
        rstep(lo_rem + NRB_T - 1, 0)
        rwait_out(e_rem, lo_rem + NRB_T - 1, 0)
        rwait_out(e_rem, lo_rem + NRB_T - 2, 1)

        plsc.subcore_barrier()


@functools.partial(jax.jit, static_argnums=())
def kernel(indices, table):
    idx_t = indices.astype(jnp.int32).T
    tab_t = table.T
    mesh = plsc.VectorSubcoreMesh(core_axis_name="c", subcore_axis_name="s")
    run = pl.kernel(
        _body,
        mesh=mesh,
        compiler_params=pltpu.CompilerParams(needs_layout_passes=False),
        out_type=jax.ShapeDtypeStruct((EMBED, SEQ, BATCH), jnp.float32),
        scratch_types=[
            pltpu.VMEM((VOCAB,), jnp.float32),
            [pltpu.VMEM((BR, BC), jnp.int32) for _ in range(2)],
            [pltpu.VMEM((BR, BC), jnp.float32) for _ in range(2)],
            [pltpu.SemaphoreType.DMA for _ in range(2)],
            [pltpu.SemaphoreType.DMA for _ in range(2)],
            [pltpu.VMEM((BR, RBC), jnp.int32) for _ in range(2)],
            [pltpu.VMEM((BR, RBC), jnp.float32) for _ in range(2)],
            pltpu.VMEM_SHARED((SEQ, QCOLS), jnp.int32),
        ],
    )
    out_t = run(idx_t, tab_t)
    return jnp.transpose(out_t, (2, 1, 0))
